# Initial kernel scaffold; baseline (speedup 1.0000x reference)
import jax
import jax.numpy as jnp
from jax.experimental import pallas as pl


def kernel(data, clusts, edge_index):
    raise NotImplementedError("write your pallas kernel here")


# speedup vs baseline: 61.4288x; 61.4288x over previous
import functools

import jax
import jax.numpy as jnp
from jax import lax
from jax.experimental import pallas as pl
from jax.experimental.pallas import tpu as pltpu
from jax.experimental.pallas import tpu_sc as plsc

N_VOX = 262144
N_CLUSTS = 16384
PTS = 16
N_EDGES = 131072
HALF = N_EDGES // 2

NC = 2
NS = 16
NW = NC * NS

BLK = 512
GRID = HALF // BLK


def _mesh():
    return plsc.VectorSubcoreMesh(
        core_axis_name="c", subcore_axis_name="s", num_cores=NC, num_subcores=NS)


def _wid():
    return lax.axis_index("s") * NC + lax.axis_index("c")


_A_ROWS = (N_CLUSTS * PTS) // 128
_A_PER_W = _A_ROWS // NW
_A_GRP = 8


def _stage_a(vox4, cidx):
    @functools.partial(
        pl.kernel,
        out_type=jax.ShapeDtypeStruct((_A_ROWS, 128, 4), jnp.float32),
        mesh=_mesh(),
        compiler_params=pltpu.CompilerParams(use_tc_tiling_on_sc=False),
        scratch_types=[
            pltpu.VMEM((_A_PER_W, 128), jnp.int32),
            pltpu.VMEM((_A_PER_W, 128, 4), jnp.float32),
            pltpu.SemaphoreType.DMA,
        ],
    )
    def ka(vox_hbm, cidx_hbm, p_hbm, idx_v, buf_v, sem):
        w = _wid()
        base = w * _A_PER_W
        pltpu.sync_copy(cidx_hbm.at[pl.ds(base, _A_PER_W)], idx_v)

        def grp(g, carry):
            descs = []
            for b in range(_A_GRP):
                r = g * _A_GRP + b
                descs.append(pltpu.async_copy(
                    vox_hbm.at[idx_v.at[r]], buf_v.at[r], sem))
            for d in descs:
                d.wait()
            return carry

        lax.fori_loop(0, _A_PER_W // _A_GRP, grp, 0)
        pltpu.sync_copy(buf_v, p_hbm.at[pl.ds(base, _A_PER_W)])

    return ka(vox4, cidx)


_B_ROWS = N_EDGES // 128
_B_PER_W = _B_ROWS // NW
_B_GRP = 4


def _stage_b(ptab, eids):
    @functools.partial(
        pl.kernel,
        out_type=jax.ShapeDtypeStruct((_B_ROWS, 128, 64), jnp.float32),
        mesh=_mesh(),
        compiler_params=pltpu.CompilerParams(use_tc_tiling_on_sc=False),
        scratch_types=[
            pltpu.VMEM((_B_PER_W, 128), jnp.int32),
            pltpu.VMEM((_B_GRP, 128, 64), jnp.float32),
            pltpu.SemaphoreType.DMA,
        ],
    )
    def kb(ptab_hbm, eids_hbm, x_hbm, idx_v, buf_v, sem):
        w = _wid()
        base = w * _B_PER_W
        pltpu.sync_copy(eids_hbm.at[pl.ds(base, _B_PER_W)], idx_v)

        def grp(g, carry):
            descs = []
            for b in range(_B_GRP):
                descs.append(pltpu.async_copy(
                    ptab_hbm.at[idx_v.at[g * _B_GRP + b]], buf_v.at[b], sem))
            for d in descs:
                d.wait()
            pltpu.sync_copy(buf_v, x_hbm.at[pl.ds(base + g * _B_GRP, _B_GRP)])
            return carry

        lax.fori_loop(0, _B_PER_W // _B_GRP, grp, 0)

    return kb(ptab, eids)


def _feats_block(x1, x2):
    f32 = jnp.float32
    i32 = jnp.int32
    hi = lax.Precision.HIGHEST

    r64 = lax.broadcasted_iota(i32, (64, 256), 0)
    col = lax.broadcasted_iota(i32, (64, 256), 1)
    p = col >> 4
    q = col & 15

    a_list, b_list = [], []
    for c in range(3):
        s1 = (r64 == p * 4 + c).astype(f32)
        s2 = (r64 == q * 4 + c).astype(f32)
        a_list.append(jnp.dot(x1, s1, precision=hi))
        b_list.append(jnp.dot(x2, s2, precision=hi))

    d0 = a_list[0] - b_list[0]
    d1 = a_list[1] - b_list[1]
    d2c = a_list[2] - b_list[2]
    d2 = d0 * d0 + d1 * d1 + d2c * d2c

    mmin = jnp.min(d2, axis=1, keepdims=True)
    lane = lax.broadcasted_iota(i32, (BLK, 256), 1)
    imin = jnp.min(jnp.where(d2 == mmin, lane, 1 << 20), axis=1, keepdims=True)
    i1 = imin >> 4
    i2 = imin & 15

    k64 = lax.broadcasted_iota(i32, (BLK, 64), 1)
    oh1 = ((k64 >> 2) == i1).astype(f32)
    oh2 = ((k64 >> 2) == i2).astype(f32)
    rsel = ((lax.broadcasted_iota(i32, (64, 4), 0) & 3)
            == lax.broadcasted_iota(i32, (64, 4), 1)).astype(f32)
    v1 = jnp.dot(x1 * oh1, rsel, precision=hi)
    v2 = jnp.dot(x2 * oh2, rsel, precision=hi)

    d3 = v1 - v2
    l2 = jnp.sum(d3 * d3, axis=1, keepdims=True)
    lend = jnp.sqrt(l2)
    pos = lend > 0
    dn = jnp.where(pos, d3 / jnp.where(pos, lend, 1.0), d3)

    wi = ((lax.broadcasted_iota(i32, (4, 9), 1) // 3)
          == lax.broadcasted_iota(i32, (4, 9), 0)).astype(f32)
    wj = ((lax.broadcasted_iota(i32, (4, 9), 1) % 3)
          == lax.broadcasted_iota(i32, (4, 9), 0)).astype(f32)
    b9 = jnp.dot(dn, wi, precision=hi) * jnp.dot(dn, wj, precision=hi)

    return jnp.concatenate(
        [v1[:, :3], v2[:, :3], dn[:, :3], lend, b9], axis=1)


def _stage_c_kernel(u_ref, x1a, x2a, x1b, x2b, o1, o2):
    u = u_ref[0, 0]
    f1 = _feats_block(x1a[...], x2a[...])
    f2 = _feats_block(x1b[...], x2b[...])
    o1[...] = f1
    flip = jnp.concatenate(
        [f1[:, 3:6], f1[:, 0:3], -f1[:, 6:9], f1[:, 9:]], axis=1)
    o2[...] = jnp.where(u > 0, flip, f2)


def _stage_c(x1, x2, u):
    blk_in = pl.BlockSpec((BLK, 64), lambda i: (i, 0))
    blk_in2 = pl.BlockSpec((BLK, 64), lambda i: (i + GRID, 0))
    blk_out = pl.BlockSpec((BLK, 19), lambda i: (i, 0))
    return pl.pallas_call(
        _stage_c_kernel,
        grid=(GRID,),
        in_specs=[
            pl.BlockSpec(memory_space=pltpu.SMEM),
            blk_in, blk_in, blk_in2, blk_in2,
        ],
        out_specs=[blk_out, blk_out],
        out_shape=[
            jax.ShapeDtypeStruct((HALF, 19), jnp.float32),
            jax.ShapeDtypeStruct((HALF, 19), jnp.float32),
        ],
    )(u, x1, x2, x1, x2)


def kernel(data, clusts, edge_index):
    vox4 = jnp.pad(data[:, 1:4].astype(jnp.float32), ((0, 0), (0, 1)))
    cidx = clusts.reshape(_A_ROWS, 128)

    p = _stage_a(vox4, cidx)
    ptab = p.reshape(N_CLUSTS, PTS * 4)

    e0 = edge_index[0].reshape(_B_ROWS, 128)
    e1 = edge_index[1].reshape(_B_ROWS, 128)
    x1 = _stage_b(ptab, e0).reshape(N_EDGES, 64)
    x2 = _stage_b(ptab, e1).reshape(N_EDGES, 64)

    und = jnp.logical_and(
        edge_index[1, 0] == edge_index[0, HALF],
        edge_index[0, 0] == edge_index[1, HALF])
    u = und.astype(jnp.int32).reshape(1, 1)

    o1, o2 = _stage_c(x1, x2, u)
    return jnp.concatenate([o1, o2], axis=0)


# speedup vs baseline: 159.2369x; 2.5922x over previous
import functools

import jax
import jax.numpy as jnp
from jax import lax
from jax.experimental import pallas as pl
from jax.experimental.pallas import tpu as pltpu
from jax.experimental.pallas import tpu_sc as plsc

N_VOX = 262144
N_CLUSTS = 16384
PTS = 16
N_EDGES = 131072
HALF = N_EDGES // 2

NC = 2
NS = 16
NW = NC * NS

BLK = 1024
GRID = HALF // BLK


def _mesh():
    return plsc.VectorSubcoreMesh(
        core_axis_name="c", subcore_axis_name="s", num_cores=NC, num_subcores=NS)


def _wid():
    return lax.axis_index("s") * NC + lax.axis_index("c")


_A_ROWS = (N_CLUSTS * PTS) // 128
_A_PER_W = _A_ROWS // NW
_A_GRP = 8


def _stage_a(vox8, cidx):
    @functools.partial(
        pl.kernel,
        out_type=jax.ShapeDtypeStruct((_A_ROWS, 128, 8), jnp.float32),
        mesh=_mesh(),
        compiler_params=pltpu.CompilerParams(use_tc_tiling_on_sc=False),
        scratch_types=[
            pltpu.VMEM((_A_PER_W, 128), jnp.int32),
            pltpu.VMEM((_A_PER_W, 128, 8), jnp.float32),
            pltpu.SemaphoreType.DMA,
        ],
    )
    def ka(vox_hbm, cidx_hbm, p_hbm, idx_v, buf_v, sem):
        w = _wid()
        base = w * _A_PER_W
        pltpu.sync_copy(cidx_hbm.at[pl.ds(base, _A_PER_W)], idx_v)

        def grp(g, carry):
            descs = []
            for b in range(_A_GRP):
                r = g * _A_GRP + b
                descs.append(pltpu.async_copy(
                    vox_hbm.at[idx_v.at[r]], buf_v.at[r], sem))
            for d in descs:
                d.wait()
            return carry

        lax.fori_loop(0, _A_PER_W // _A_GRP, grp, 0)
        pltpu.sync_copy(buf_v, p_hbm.at[pl.ds(base, _A_PER_W)])

    return ka(vox8, cidx)


_A2_BLK = 512


def _stage_a2_kernel(x_ref, o_ref):
    i32 = jnp.int32
    r = lax.broadcasted_iota(i32, (128, 64), 0)
    s = lax.broadcasted_iota(i32, (128, 64), 1)
    sel = (r == ((s >> 2) * 8 + (s & 3))).astype(jnp.float32)
    o_ref[...] = jnp.dot(x_ref[...], sel, precision=lax.Precision.HIGHEST)


def _stage_a2(p8):
    return pl.pallas_call(
        _stage_a2_kernel,
        grid=(N_CLUSTS // _A2_BLK,),
        in_specs=[pl.BlockSpec((_A2_BLK, 128), lambda i: (i, 0))],
        out_specs=pl.BlockSpec((_A2_BLK, 64), lambda i: (i, 0)),
        out_shape=jax.ShapeDtypeStruct((N_CLUSTS, 64), jnp.float32),
    )(p8)


_B_ROWS = (2 * N_EDGES) // 128
_B_PER_W = _B_ROWS // NW
_B_GRP = 4


def _stage_b(ptab, eids):
    @functools.partial(
        pl.kernel,
        out_type=jax.ShapeDtypeStruct((_B_ROWS, 128, 64), jnp.float32),
        mesh=_mesh(),
        compiler_params=pltpu.CompilerParams(use_tc_tiling_on_sc=False),
        scratch_types=[
            pltpu.VMEM((_B_PER_W, 128), jnp.int32),
            pltpu.VMEM((_B_GRP, 128, 64), jnp.float32),
            pltpu.SemaphoreType.DMA,
        ],
    )
    def kb(ptab_hbm, eids_hbm, x_hbm, idx_v, buf_v, sem):
        w = _wid()
        base = w * _B_PER_W
        pltpu.sync_copy(eids_hbm.at[pl.ds(base, _B_PER_W)], idx_v)

        def grp(g, carry):
            descs = []
            for b in range(_B_GRP):
                descs.append(pltpu.async_copy(
                    ptab_hbm.at[idx_v.at[g * _B_GRP + b]], buf_v.at[b], sem))
            for d in descs:
                d.wait()
            pltpu.sync_copy(buf_v, x_hbm.at[pl.ds(base + g * _B_GRP, _B_GRP)])
            return carry

        lax.fori_loop(0, _B_PER_W // _B_GRP, grp, 0)

    return kb(ptab, eids)


def _feats_block(x1, x2):
    f32 = jnp.float32
    i32 = jnp.int32
    hi = lax.Precision.HIGHEST

    r = lax.broadcasted_iota(i32, (128, 128), 0)
    s = lax.broadcasted_iota(i32, (128, 128), 1)
    perm = (((r & 63) == ((s & 15) * 4 + ((s >> 4) & 3)))
            & ((r >> 6) == (s >> 6))).astype(f32)
    xt = jnp.transpose(
        jnp.dot(jnp.concatenate([x1, x2], axis=1), perm, precision=hi))
    x1c = [xt[0:16], xt[16:32], xt[32:48]]
    x2c = [xt[64:80], xt[80:96], xt[96:112]]

    x1b = [c.astype(jnp.bfloat16).astype(f32) for c in x1c]
    x2b = [c.astype(jnp.bfloat16).astype(f32) for c in x2c]
    n1 = (x1c[0] * x1c[0] + x1c[1] * x1c[1]) + x1c[2] * x1c[2]
    n2 = (x2c[0] * x2c[0] + x2c[1] * x2c[1]) + x2c[2] * x2c[2]

    m = jnp.full((16, BLK), jnp.inf, f32)
    im = jnp.zeros((16, BLK), i32)
    for p in range(16):
        cross = ((x1b[0][p:p + 1] * x2b[0] + x1b[1][p:p + 1] * x2b[1])
                 + x1b[2][p:p + 1] * x2b[2])
        d2 = (n1[p:p + 1] + n2) - 2.0 * cross
        upd = d2 < m
        m = jnp.where(upd, d2, m)
        im = jnp.where(upd, p, im)

    qio = lax.broadcasted_iota(i32, (16, BLK), 0)
    flat = im * 16 + qio
    qm = jnp.min(m, axis=0, keepdims=True)
    bestflat = jnp.min(jnp.where(m == qm, flat, 1 << 20), axis=0, keepdims=True)
    i1 = bestflat >> 4
    i2 = bestflat & 15

    oh1 = (qio == i1).astype(f32)
    oh2 = (qio == i2).astype(f32)
    v1 = [jnp.sum(oh1 * x1c[c], axis=0, keepdims=True) for c in range(3)]
    v2 = [jnp.sum(oh2 * x2c[c], axis=0, keepdims=True) for c in range(3)]

    d3 = [v1[c] - v2[c] for c in range(3)]
    l2 = d3[0] * d3[0] + d3[1] * d3[1] + d3[2] * d3[2]
    lend = jnp.sqrt(l2)
    pos = lend > 0
    safe = jnp.where(pos, lend, 1.0)
    dn = [jnp.where(pos, d3[c] / safe, d3[c]) for c in range(3)]

    b9 = [dn[i] * dn[j] for i in range(3) for j in range(3)]
    return jnp.concatenate(v1 + v2 + dn + [lend] + b9, axis=0)


def _stage_c_kernel(u_ref, x1a, x2a, x1b, x2b, o1, o2):
    u = u_ref[0, 0]
    f1 = _feats_block(x1a[...], x2a[...])
    f2 = _feats_block(x1b[...], x2b[...])
    o1[...] = f1
    flip = jnp.concatenate([f1[3:6], f1[0:3], -f1[6:9], f1[9:]], axis=0)
    o2[...] = jnp.where(u > 0, flip, f2)


def _stage_c(xall, u):
    nb = N_EDGES // BLK
    blk_x1a = pl.BlockSpec((BLK, 64), lambda i: (i, 0))
    blk_x2a = pl.BlockSpec((BLK, 64), lambda i: (i + nb, 0))
    blk_x1b = pl.BlockSpec((BLK, 64), lambda i: (i + GRID, 0))
    blk_x2b = pl.BlockSpec((BLK, 64), lambda i: (i + nb + GRID, 0))
    blk_out = pl.BlockSpec((19, BLK), lambda i: (0, i))
    return pl.pallas_call(
        _stage_c_kernel,
        grid=(GRID,),
        in_specs=[
            pl.BlockSpec(memory_space=pltpu.SMEM),
            blk_x1a, blk_x2a, blk_x1b, blk_x2b,
        ],
        out_specs=[blk_out, blk_out],
        out_shape=[
            jax.ShapeDtypeStruct((19, HALF), jnp.float32),
            jax.ShapeDtypeStruct((19, HALF), jnp.float32),
        ],
    )(u, xall, xall, xall, xall)


def kernel(data, clusts, edge_index):
    vox8 = jnp.pad(data[:, 1:4].astype(jnp.float32), ((0, 0), (0, 5)))
    cidx = clusts.reshape(_A_ROWS, 128)

    p8 = _stage_a(vox8, cidx)
    ptab = _stage_a2(p8.reshape(N_CLUSTS, PTS * 8))

    eids = edge_index.reshape(_B_ROWS, 128)
    xall = _stage_b(ptab, eids).reshape(2 * N_EDGES, 64)

    und = jnp.logical_and(
        edge_index[1, 0] == edge_index[0, HALF],
        edge_index[0, 0] == edge_index[1, HALF])
    u = und.astype(jnp.int32).reshape(1, 1)

    o1, o2 = _stage_c(xall, u)
    return jnp.concatenate([o1, o2], axis=1).T


# speedup vs baseline: 171.3226x; 1.0759x over previous
import functools

import jax
import jax.numpy as jnp
from jax import lax
from jax.experimental import pallas as pl
from jax.experimental.pallas import tpu as pltpu
from jax.experimental.pallas import tpu_sc as plsc

N_VOX = 262144
N_CLUSTS = 16384
PTS = 16
N_EDGES = 131072
HALF = N_EDGES // 2

NC = 2
NS = 16
NW = NC * NS

BLK = 1024
GRID = HALF // BLK


def _mesh():
    return plsc.VectorSubcoreMesh(
        core_axis_name="c", subcore_axis_name="s", num_cores=NC, num_subcores=NS)


def _wid():
    return lax.axis_index("s") * NC + lax.axis_index("c")


_A_ROWS = (N_CLUSTS * PTS) // 128
_A_PER_W = _A_ROWS // NW
_A_GRP = 8


def _stage_a(vox8, cidx):
    @functools.partial(
        pl.kernel,
        out_type=jax.ShapeDtypeStruct((_A_ROWS, 128, 8), jnp.float32),
        mesh=_mesh(),
        compiler_params=pltpu.CompilerParams(use_tc_tiling_on_sc=False),
        scratch_types=[
            pltpu.VMEM((_A_PER_W, 128), jnp.int32),
            pltpu.VMEM((_A_PER_W, 128, 8), jnp.float32),
            pltpu.SemaphoreType.DMA,
        ],
    )
    def ka(vox_hbm, cidx_hbm, p_hbm, idx_v, buf_v, sem):
        w = _wid()
        base = w * _A_PER_W
        pltpu.sync_copy(cidx_hbm.at[pl.ds(base, _A_PER_W)], idx_v)

        def grp(g, carry):
            descs = []
            for b in range(_A_GRP):
                r = g * _A_GRP + b
                descs.append(pltpu.async_copy(
                    vox_hbm.at[idx_v.at[r]], buf_v.at[r], sem))
            for d in descs:
                d.wait()
            return carry

        lax.fori_loop(0, _A_PER_W // _A_GRP, grp, 0)
        pltpu.sync_copy(buf_v, p_hbm.at[pl.ds(base, _A_PER_W)])

    return ka(vox8, cidx)


_A2_BLK = 512


def _stage_a2_kernel(x_ref, o_ref):
    i32 = jnp.int32
    r = lax.broadcasted_iota(i32, (128, 64), 0)
    s = lax.broadcasted_iota(i32, (128, 64), 1)
    sel = ((r == ((s & 15) * 8 + (s >> 4))) & ((s >> 4) < 3)).astype(jnp.float32)
    o_ref[...] = jnp.dot(x_ref[...], sel, precision=lax.Precision.HIGHEST)


def _stage_a2(p8):
    return pl.pallas_call(
        _stage_a2_kernel,
        grid=(N_CLUSTS // _A2_BLK,),
        in_specs=[pl.BlockSpec((_A2_BLK, 128), lambda i: (i, 0))],
        out_specs=pl.BlockSpec((_A2_BLK, 64), lambda i: (i, 0)),
        out_shape=jax.ShapeDtypeStruct((N_CLUSTS, 64), jnp.float32),
    )(p8)


_B_ROWS = (2 * N_EDGES) // 128
_B_PER_W = _B_ROWS // NW
_B_GRP = 4


def _stage_b(ptab, eids):
    @functools.partial(
        pl.kernel,
        out_type=jax.ShapeDtypeStruct((_B_ROWS, 128, 64), jnp.float32),
        mesh=_mesh(),
        compiler_params=pltpu.CompilerParams(use_tc_tiling_on_sc=False),
        scratch_types=[
            pltpu.VMEM((_B_PER_W, 128), jnp.int32),
            pltpu.VMEM((_B_GRP, 128, 64), jnp.float32),
            pltpu.SemaphoreType.DMA,
        ],
    )
    def kb(ptab_hbm, eids_hbm, x_hbm, idx_v, buf_v, sem):
        w = _wid()
        base = w * _B_PER_W
        pltpu.sync_copy(eids_hbm.at[pl.ds(base, _B_PER_W)], idx_v)

        def grp(g, carry):
            descs = []
            for b in range(_B_GRP):
                descs.append(pltpu.async_copy(
                    ptab_hbm.at[idx_v.at[g * _B_GRP + b]], buf_v.at[b], sem))
            for d in descs:
                d.wait()
            pltpu.sync_copy(buf_v, x_hbm.at[pl.ds(base + g * _B_GRP, _B_GRP)])
            return carry

        lax.fori_loop(0, _B_PER_W // _B_GRP, grp, 0)

    return kb(ptab, eids)


def _feats_block(x1, x2):
    f32 = jnp.float32
    i32 = jnp.int32

    xt = jnp.transpose(jnp.concatenate([x1, x2], axis=1))
    x1c = [xt[0:16], xt[16:32], xt[32:48]]
    x2c = [xt[64:80], xt[80:96], xt[96:112]]

    x1b = [c.astype(jnp.bfloat16).astype(f32) for c in x1c]
    x2b = [c.astype(jnp.bfloat16).astype(f32) for c in x2c]
    n1 = (x1c[0] * x1c[0] + x1c[1] * x1c[1]) + x1c[2] * x1c[2]
    n2 = (x2c[0] * x2c[0] + x2c[1] * x2c[1]) + x2c[2] * x2c[2]

    m = jnp.full((16, BLK), jnp.inf, f32)
    im = jnp.zeros((16, BLK), i32)
    for p in range(16):
        cross = ((x1b[0][p:p + 1] * x2b[0] + x1b[1][p:p + 1] * x2b[1])
                 + x1b[2][p:p + 1] * x2b[2])
        d2 = (n1[p:p + 1] + n2) - 2.0 * cross
        upd = d2 < m
        m = jnp.where(upd, d2, m)
        im = jnp.where(upd, p, im)

    qio = lax.broadcasted_iota(i32, (16, BLK), 0)
    flat = im * 16 + qio
    qm = jnp.min(m, axis=0, keepdims=True)
    bestflat = jnp.min(jnp.where(m == qm, flat, 1 << 20), axis=0, keepdims=True)
    i1 = bestflat >> 4
    i2 = bestflat & 15

    oh1 = (qio == i1).astype(f32)
    oh2 = (qio == i2).astype(f32)
    v1 = [jnp.sum(oh1 * x1c[c], axis=0, keepdims=True) for c in range(3)]
    v2 = [jnp.sum(oh2 * x2c[c], axis=0, keepdims=True) for c in range(3)]

    d3 = [v1[c] - v2[c] for c in range(3)]
    l2 = d3[0] * d3[0] + d3[1] * d3[1] + d3[2] * d3[2]
    lend = jnp.sqrt(l2)
    pos = lend > 0
    safe = jnp.where(pos, lend, 1.0)
    dn = [jnp.where(pos, d3[c] / safe, d3[c]) for c in range(3)]

    b9 = [dn[i] * dn[j] for i in range(3) for j in range(3)]
    return jnp.concatenate(v1 + v2 + dn + [lend] + b9, axis=0)


def _stage_c_kernel(u_ref, x1a, x2a, x1b, x2b, o1, o2):
    u = u_ref[0, 0]
    f1 = _feats_block(x1a[...], x2a[...])
    f2 = _feats_block(x1b[...], x2b[...])
    o1[...] = f1
    flip = jnp.concatenate([f1[3:6], f1[0:3], -f1[6:9], f1[9:]], axis=0)
    o2[...] = jnp.where(u > 0, flip, f2)


def _stage_c(xall, u):
    nb = N_EDGES // BLK
    blk_x1a = pl.BlockSpec((BLK, 64), lambda i: (i, 0))
    blk_x2a = pl.BlockSpec((BLK, 64), lambda i: (i + nb, 0))
    blk_x1b = pl.BlockSpec((BLK, 64), lambda i: (i + GRID, 0))
    blk_x2b = pl.BlockSpec((BLK, 64), lambda i: (i + nb + GRID, 0))
    blk_out = pl.BlockSpec((19, BLK), lambda i: (0, i))
    return pl.pallas_call(
        _stage_c_kernel,
        grid=(GRID,),
        in_specs=[
            pl.BlockSpec(memory_space=pltpu.SMEM),
            blk_x1a, blk_x2a, blk_x1b, blk_x2b,
        ],
        out_specs=[blk_out, blk_out],
        out_shape=[
            jax.ShapeDtypeStruct((19, HALF), jnp.float32),
            jax.ShapeDtypeStruct((19, HALF), jnp.float32),
        ],
    )(u, xall, xall, xall, xall)


def kernel(data, clusts, edge_index):
    vox8 = jnp.pad(data[:, 1:4].astype(jnp.float32), ((0, 0), (0, 5)))
    cidx = clusts.reshape(_A_ROWS, 128)

    p8 = _stage_a(vox8, cidx)
    ptab = _stage_a2(p8.reshape(N_CLUSTS, PTS * 8))

    eids = edge_index.reshape(_B_ROWS, 128)
    xall = _stage_b(ptab, eids).reshape(2 * N_EDGES, 64)

    und = jnp.logical_and(
        edge_index[1, 0] == edge_index[0, HALF],
        edge_index[0, 0] == edge_index[1, HALF])
    u = und.astype(jnp.int32).reshape(1, 1)

    o1, o2 = _stage_c(xall, u)
    return jnp.concatenate([o1, o2], axis=1).T
